# Initial kernel scaffold; baseline (speedup 1.0000x reference)
import jax
import jax.numpy as jnp
from jax.experimental import pallas as pl


def kernel(input_embedding, W1, b1, ln_g, ln_b, W2, b2, Wl, bl, Wr):
    raise NotImplementedError("write your pallas kernel here")


# speedup vs baseline: 1.9236x; 1.9236x over previous
import functools
import jax
import jax.numpy as jnp
from jax.experimental import pallas as pl

C_IN = 26
C_OUT = 8
HID = 128
B = 1024

ROW_BLK = 2048


def _stage_a_body(x_ref, w1t_ref, b1_ref, out_ref):
    x = x_ref[:]
    h = jnp.dot(x, w1t_ref[:], preferred_element_type=jnp.float32) + b1_ref[:]
    out_ref[:] = jnp.maximum(h, 0.0)


def _stage_b_body(f_ref, s_ref, fit_ref, t_ref, pos_ref, deg_ref):
    fi = f_ref[:]
    f = fi
    c_iota = jax.lax.broadcasted_iota(jnp.int32, (B, C_IN), 1).astype(jnp.float32)
    rank = jnp.zeros((B, C_IN), jnp.float32)
    for cp in range(C_IN):
        col = f[:, cp:cp + 1]
        gt = (col > f).astype(jnp.float32)
        tie = jnp.logical_and(col == f, cp < c_iota).astype(jnp.float32)
        rank = rank + gt + tie
    t = (rank < float(C_OUT)).astype(jnp.float32)
    pos = jnp.zeros((B, C_IN), jnp.float32)
    for cp in range(C_IN):
        pos = pos + t[:, cp:cp + 1] * (cp < c_iota).astype(jnp.float32)
    g = jax.lax.dot_general(t, t, (((1,), (1,)), ((), ())),
                            preferred_element_type=jnp.float32)
    s = (g >= 1.5).astype(jnp.float32)
    s_ref[:] = s
    fit_ref[:] = fi * t
    t_ref[:] = t
    pos_ref[:] = pos
    deg_ref[:] = jnp.dot(s, t, preferred_element_type=jnp.float32)


def _stage_c_body(s_ref, x_ref, fit_ref, z_ref):
    y = x_ref[0] * fit_ref[0]
    z_ref[:] = jnp.dot(s_ref[:], y, preferred_element_type=jnp.float32)


def _stage_d_body(z_ref, x_ref, fit_ref, t_ref, deg_ref, pos_ref,
                  wlt_ref, bl_ref, wrt_ref, out_ref):
    tc = t_ref[0]
    degc = deg_ref[0]
    aggr = z_ref[:] / jnp.maximum(degc, 1.0) * tc
    ym = x_ref[0] * fit_ref[0]
    o = (jnp.dot(aggr, wlt_ref[0], preferred_element_type=jnp.float32)
         + bl_ref[0]
         + jnp.dot(ym, wrt_ref[0], preferred_element_type=jnp.float32))
    o = jnp.maximum(o, 0.0)
    cnt = jnp.maximum(jnp.sum(tc) * float(HID), 1.0)
    mean = jnp.sum(o * tc) / cnt
    dv = o - mean
    var = jnp.sum(dv * dv * tc) / cnt
    onm = dv * jax.lax.rsqrt(var + 1e-5) * tc
    posc = pos_ref[0]

    @pl.when(pl.program_id(0) == 0)
    def _():
        out_ref[:] = jnp.zeros_like(out_ref)

    for k in range(C_OUT):
        mk = (posc == float(k)).astype(jnp.float32)
        out_ref[:, k * HID:(k + 1) * HID] += onm * mk


def kernel(input_embedding, W1, b1, ln_g, ln_b, W2, b2, Wl, bl, Wr):
    x = input_embedding.astype(jnp.float32)
    N = B * C_IN

    x2d = x.reshape(N, HID)
    h2d = pl.pallas_call(
        _stage_a_body,
        grid=(N // ROW_BLK,),
        in_specs=[
            pl.BlockSpec((ROW_BLK, HID), lambda i: (i, 0)),
            pl.BlockSpec((HID, HID), lambda i: (0, 0)),
            pl.BlockSpec((1, HID), lambda i: (0, 0)),
        ],
        out_specs=pl.BlockSpec((ROW_BLK, HID), lambda i: (i, 0)),
        out_shape=jax.ShapeDtypeStruct((N, HID), jnp.float32),
    )(x2d, W1.T, b1.reshape(1, HID))
    h = h2d.reshape(B, C_IN, HID)
    mu = h.mean(-1, keepdims=True)
    var = h.var(-1, keepdims=True)
    hn = (h - mu) / jnp.sqrt(var + 1e-5) * ln_g + ln_b
    f = (jnp.einsum('bch,oh->bco', hn, W2) + b2)[..., 0]
    f = jax.nn.softmax(f, axis=1)

    s_mat, fit, t, pos, deg = pl.pallas_call(
        _stage_b_body,
        out_shape=[
            jax.ShapeDtypeStruct((B, B), jnp.float32),
            jax.ShapeDtypeStruct((B, C_IN), jnp.float32),
            jax.ShapeDtypeStruct((B, C_IN), jnp.float32),
            jax.ShapeDtypeStruct((B, C_IN), jnp.float32),
            jax.ShapeDtypeStruct((B, C_IN), jnp.float32),
        ],
    )(f)

    xt = jnp.transpose(x, (1, 0, 2))
    fit_e = fit.T.reshape(C_IN, B, 1)
    t_e = t.T.reshape(C_IN, B, 1)
    pos_e = pos.T.reshape(C_IN, B, 1)
    deg_e = deg.T.reshape(C_IN, B, 1)

    z = pl.pallas_call(
        _stage_c_body,
        grid=(C_IN,),
        in_specs=[
            pl.BlockSpec((B, B), lambda c: (0, 0)),
            pl.BlockSpec((1, B, HID), lambda c: (c, 0, 0)),
            pl.BlockSpec((1, B, 1), lambda c: (c, 0, 0)),
        ],
        out_specs=pl.BlockSpec((B, HID), lambda c: (0, c)),
        out_shape=jax.ShapeDtypeStruct((B, C_IN * HID), jnp.float32),
    )(s_mat, xt, fit_e)

    out2d = pl.pallas_call(
        _stage_d_body,
        grid=(C_IN,),
        in_specs=[
            pl.BlockSpec((B, HID), lambda c: (0, c)),
            pl.BlockSpec((1, B, HID), lambda c: (c, 0, 0)),
            pl.BlockSpec((1, B, 1), lambda c: (c, 0, 0)),
            pl.BlockSpec((1, B, 1), lambda c: (c, 0, 0)),
            pl.BlockSpec((1, B, 1), lambda c: (c, 0, 0)),
            pl.BlockSpec((1, B, 1), lambda c: (c, 0, 0)),
            pl.BlockSpec((1, HID, HID), lambda c: (c, 0, 0)),
            pl.BlockSpec((1, 1, HID), lambda c: (c, 0, 0)),
            pl.BlockSpec((1, HID, HID), lambda c: (c, 0, 0)),
        ],
        out_specs=pl.BlockSpec((B, C_OUT * HID), lambda c: (0, 0)),
        out_shape=jax.ShapeDtypeStruct((B, C_OUT * HID), jnp.float32),
    )(z, xt, fit_e, t_e, deg_e, pos_e,
      jnp.transpose(Wl, (0, 2, 1)), bl.reshape(C_IN, 1, HID),
      jnp.transpose(Wr, (0, 2, 1)))

    return out2d.reshape(B, C_OUT, HID)


# speedup vs baseline: 1.9732x; 1.0258x over previous
import functools
import jax
import jax.numpy as jnp
from jax.experimental import pallas as pl

C_IN = 26
C_OUT = 8
HID = 128
B = 1024

ROW_BLK = 2048


def _stage_a_body(x_ref, w1t_ref, b1_ref, out_ref):
    x = x_ref[:]
    h = jnp.dot(x, w1t_ref[:], preferred_element_type=jnp.float32) + b1_ref[:]
    out_ref[:] = jnp.maximum(h, 0.0)


def _stage_b_body(f_ref, s_ref, fit_ref, key_ref, deg_ref):
    fi = f_ref[:]
    f = fi
    c_iota = jax.lax.broadcasted_iota(jnp.int32, (B, C_IN), 1).astype(jnp.float32)
    rank = jnp.zeros((B, C_IN), jnp.float32)
    for cp in range(C_IN):
        col = f[:, cp:cp + 1]
        gt = (col > f).astype(jnp.float32)
        tie = jnp.logical_and(col == f, cp < c_iota).astype(jnp.float32)
        rank = rank + gt + tie
    t = (rank < float(C_OUT)).astype(jnp.float32)
    pos = jnp.zeros((B, C_IN), jnp.float32)
    for cp in range(C_IN):
        pos = pos + t[:, cp:cp + 1] * (cp < c_iota).astype(jnp.float32)
    g = jax.lax.dot_general(t, t, (((1,), (1,)), ((), ())),
                            preferred_element_type=jnp.float32)
    s = (g >= 1.5).astype(jnp.float32)
    s_ref[:] = s
    fit_ref[:] = fi * t
    key_ref[:] = jnp.where(t > 0.0, pos, -1.0)
    deg_ref[:] = jnp.dot(s, t, preferred_element_type=jnp.float32)


def _stage_cd_body(s_ref, x_ref, fit_ref, key_ref, deg_ref,
                   wlt_ref, bl_ref, wrt_ref, out_ref):
    keyc = key_ref[0]
    degc = deg_ref[0]
    y = x_ref[:] * fit_ref[0]
    z = jnp.dot(s_ref[:], y, preferred_element_type=jnp.float32)
    tc = (keyc >= 0.0).astype(jnp.float32)
    aggr = z / jnp.maximum(degc, 1.0) * tc
    o = (jnp.dot(aggr, wlt_ref[0], preferred_element_type=jnp.float32)
         + bl_ref[0]
         + jnp.dot(y, wrt_ref[0], preferred_element_type=jnp.float32))
    o = jnp.maximum(o, 0.0)
    cnt = jnp.maximum(jnp.sum(tc) * float(HID), 1.0)
    mean = jnp.sum(o * tc) / cnt
    dv = o - mean
    var = jnp.sum(dv * dv * tc) / cnt
    onm = dv * jax.lax.rsqrt(var + 1e-5)

    @pl.when(pl.program_id(0) == 0)
    def _():
        out_ref[:] = jnp.zeros_like(out_ref)

    for k in range(C_OUT):
        mk = (keyc == float(k)).astype(jnp.float32)
        out_ref[:, k * HID:(k + 1) * HID] += onm * mk


def kernel(input_embedding, W1, b1, ln_g, ln_b, W2, b2, Wl, bl, Wr):
    x = input_embedding.astype(jnp.float32)
    N = B * C_IN

    x2d = x.reshape(N, HID)
    h2d = pl.pallas_call(
        _stage_a_body,
        grid=(N // ROW_BLK,),
        in_specs=[
            pl.BlockSpec((ROW_BLK, HID), lambda i: (i, 0)),
            pl.BlockSpec((HID, HID), lambda i: (0, 0)),
            pl.BlockSpec((1, HID), lambda i: (0, 0)),
        ],
        out_specs=pl.BlockSpec((ROW_BLK, HID), lambda i: (i, 0)),
        out_shape=jax.ShapeDtypeStruct((N, HID), jnp.float32),
    )(x2d, W1.T, b1.reshape(1, HID))
    h = h2d.reshape(B, C_IN, HID)
    mu = h.mean(-1, keepdims=True)
    var = h.var(-1, keepdims=True)
    hn = (h - mu) / jnp.sqrt(var + 1e-5) * ln_g + ln_b
    f = (jnp.einsum('bch,oh->bco', hn, W2) + b2)[..., 0]
    f = jax.nn.softmax(f, axis=1)

    s_mat, fit, key, deg = pl.pallas_call(
        _stage_b_body,
        out_shape=[
            jax.ShapeDtypeStruct((B, B), jnp.float32),
            jax.ShapeDtypeStruct((B, C_IN), jnp.float32),
            jax.ShapeDtypeStruct((B, C_IN), jnp.float32),
            jax.ShapeDtypeStruct((B, C_IN), jnp.float32),
        ],
    )(f)

    x2dch = x.reshape(B, C_IN * HID)
    fit_e = fit.T.reshape(C_IN, B, 1)
    key_e = key.T.reshape(C_IN, B, 1)
    deg_e = deg.T.reshape(C_IN, B, 1)

    out2d = pl.pallas_call(
        _stage_cd_body,
        grid=(C_IN,),
        in_specs=[
            pl.BlockSpec((B, B), lambda c: (0, 0)),
            pl.BlockSpec((B, HID), lambda c: (0, c)),
            pl.BlockSpec((1, B, 1), lambda c: (c, 0, 0)),
            pl.BlockSpec((1, B, 1), lambda c: (c, 0, 0)),
            pl.BlockSpec((1, B, 1), lambda c: (c, 0, 0)),
            pl.BlockSpec((1, HID, HID), lambda c: (c, 0, 0)),
            pl.BlockSpec((1, 1, HID), lambda c: (c, 0, 0)),
            pl.BlockSpec((1, HID, HID), lambda c: (c, 0, 0)),
        ],
        out_specs=pl.BlockSpec((B, C_OUT * HID), lambda c: (0, 0)),
        out_shape=jax.ShapeDtypeStruct((B, C_OUT * HID), jnp.float32),
    )(s_mat, x2dch, fit_e, key_e, deg_e,
      jnp.transpose(Wl, (0, 2, 1)), bl.reshape(C_IN, 1, HID),
      jnp.transpose(Wr, (0, 2, 1)))

    return out2d.reshape(B, C_OUT, HID)


# speedup vs baseline: 2.5915x; 1.3134x over previous
import jax
import jax.numpy as jnp
from jax.experimental import pallas as pl

C_IN = 26
C_OUT = 8
HID = 128
B = 1024

A_BLK = 128


def _stage_a_body(x_ref, w1t_ref, b1_ref, h_ref):
    w1t = w1t_ref[:]
    b1 = b1_ref[:]
    for c in range(C_IN):
        h = jnp.dot(x_ref[:, c, :], w1t, preferred_element_type=jnp.float32)
        h_ref[:, c, :] = jnp.maximum(h + b1, 0.0)


def _stage_bcd_body(f_ref, x_ref, wlt_ref, bl_ref, wrt_ref, out_ref):
    fi = f_ref[:]
    f = fi
    c_iota = jax.lax.broadcasted_iota(jnp.int32, (B, C_IN), 1).astype(jnp.float32)
    rank = jnp.zeros((B, C_IN), jnp.float32)
    for cp in range(C_IN):
        col = f[:, cp:cp + 1]
        gt = (col > f).astype(jnp.float32)
        tie = jnp.logical_and(col == f, cp < c_iota).astype(jnp.float32)
        rank = rank + gt + tie
    t = (rank < float(C_OUT)).astype(jnp.float32)
    pos = jnp.zeros((B, C_IN), jnp.float32)
    for cp in range(C_IN):
        pos = pos + t[:, cp:cp + 1] * (cp < c_iota).astype(jnp.float32)
    key = jnp.where(t > 0.0, pos, -1.0)
    g = jax.lax.dot_general(t, t, (((1,), (1,)), ((), ())),
                            preferred_element_type=jnp.float32)
    s = (g >= 1.5).astype(jnp.float32)
    deg = jnp.dot(s, t, preferred_element_type=jnp.float32)
    fit = fi * t

    out_ref[:] = jnp.zeros_like(out_ref)
    for c in range(C_IN):
        keyc = key[:, c:c + 1]
        degc = deg[:, c:c + 1]
        tc = (keyc >= 0.0).astype(jnp.float32)
        y = x_ref[:, c, :] * fit[:, c:c + 1]
        z = jnp.dot(s, y, preferred_element_type=jnp.float32)
        aggr = z / jnp.maximum(degc, 1.0) * tc
        o = (jnp.dot(aggr, wlt_ref[c], preferred_element_type=jnp.float32)
             + bl_ref[c]
             + jnp.dot(y, wrt_ref[c], preferred_element_type=jnp.float32))
        o = jnp.maximum(o, 0.0)
        cnt = jnp.maximum(jnp.sum(tc) * float(HID), 1.0)
        mean = jnp.sum(o * tc) / cnt
        dv = o - mean
        var = jnp.sum(dv * dv * tc) / cnt
        onm = dv * jax.lax.rsqrt(var + 1e-5)
        for k in range(C_OUT):
            mk = (keyc == float(k)).astype(jnp.float32)
            out_ref[:, k * HID:(k + 1) * HID] += onm * mk


def kernel(input_embedding, W1, b1, ln_g, ln_b, W2, b2, Wl, bl, Wr):
    x = input_embedding.astype(jnp.float32)

    h = pl.pallas_call(
        _stage_a_body,
        grid=(B // A_BLK,),
        in_specs=[
            pl.BlockSpec((A_BLK, C_IN, HID), lambda i: (i, 0, 0)),
            pl.BlockSpec((HID, HID), lambda i: (0, 0)),
            pl.BlockSpec((1, HID), lambda i: (0, 0)),
        ],
        out_specs=pl.BlockSpec((A_BLK, C_IN, HID), lambda i: (i, 0, 0)),
        out_shape=jax.ShapeDtypeStruct((B, C_IN, HID), jnp.float32),
    )(x, W1.T, b1.reshape(1, HID))

    mu = h.mean(-1, keepdims=True)
    var = h.var(-1, keepdims=True)
    hn = (h - mu) / jnp.sqrt(var + 1e-5) * ln_g + ln_b
    f = (jnp.einsum('bch,oh->bco', hn, W2) + b2)[..., 0]
    f = jax.nn.softmax(f, axis=1)

    out2d = pl.pallas_call(
        _stage_bcd_body,
        out_shape=jax.ShapeDtypeStruct((B, C_OUT * HID), jnp.float32),
    )(f, x, jnp.transpose(Wl, (0, 2, 1)), bl.reshape(C_IN, 1, HID),
      jnp.transpose(Wr, (0, 2, 1)))

    return out2d.reshape(B, C_OUT, HID)
